# baseline (speedup 1.0000x reference)
import functools
import jax
import jax.numpy as jnp
from jax import lax
from jax.experimental import pallas as pl
from jax.experimental.pallas import tpu as pltpu
from jax.experimental.pallas import tpu_sc as plsc

NC = 2
NS = 16
L = 16


def _sc_scatter_fn(N, E, D, B, x_hbm, pk_hbm, out_hbm,
                   pk, dv, rows, acc_sh, sems):
    NW = NC * NS
    e_per_tile = E // NW
    nbatch = e_per_tile // B
    nquad = (nbatch - 1) // 4
    nchunk_rows = N // B
    nround = pl.cdiv(nchunk_rows, NS)
    nchunk = D // L

    c = lax.axis_index("c")
    s = lax.axis_index("s")
    wid = s * NC + c

    zeros = jnp.zeros((L,), jnp.float32)

    def zero_body(t, _):
        r = t // nchunk
        j = t % nchunk
        rows[0][r, pl.ds(j * L, L)] = zeros
        return _

    lax.fori_loop(0, B * nchunk, zero_body, None)

    def acczero_body(t, _):
        chunk = s + NS * t

        @pl.when(chunk < nchunk_rows)
        def _():
            pltpu.sync_copy(rows[0], acc_sh.at[pl.ds(chunk * B, B)])

        return _

    lax.fori_loop(0, nround, acczero_body, None)
    plsc.subcore_barrier()

    bbase = wid * nbatch
    spk, sg, ss = sems

    def eload(b, k):
        pltpu.async_copy(pk_hbm.at[bbase + b], pk[k], spk[k])

    def eload_wait(k):
        pltpu.make_async_copy(pk_hbm.at[0], pk[k], spk[k]).wait()

    def gather(k):
        pltpu.async_copy(x_hbm.at[pk[k].at[0]], rows[k], sg[k])

    def gather_wait(k):
        pltpu.make_async_copy(x_hbm.at[pl.ds(0, B)], rows[k], sg[k]).wait()

    def scale(k):
        def scale_body(g, _):
            wgrp = lax.bitcast_convert_type(
                pk[k][2, pl.ds(g * L, L)], jnp.float32)
            for i in range(L):
                e = g * L + i
                wvec = lax.gather(
                    wgrp, jnp.full((L, 1), i, jnp.int32),
                    lax.GatherDimensionNumbers(
                        offset_dims=(), collapsed_slice_dims=(0,),
                        start_index_map=(0,)),
                    (1,), mode=lax.GatherScatterMode.PROMISE_IN_BOUNDS)
                for j in range(nchunk):
                    sl = pl.ds(j * L, L)
                    rows[k][e, sl] = rows[k][e, sl] * wvec
            return _

        lax.fori_loop(0, B // L, scale_body, None)

    def scatter(k):
        for q in range(B // L):
            sl = pl.ds(q * L, L)
            dv[k][sl] = pk[k][1, sl]
        pltpu.async_copy(rows[k], acc_sh.at[dv[k]], ss[k], add=True)

    def scatter_wait(k):
        pltpu.make_async_copy(rows[k], acc_sh.at[pl.ds(0, B)], ss[k]).wait()

    def step(b, i):
        jg = (i + 2) % 4
        js = (i + 3) % 4

        @pl.when(b >= 2)
        def _():
            scatter_wait(jg)

        @pl.when(b + 2 < nbatch)
        def _():
            eload_wait(jg)
            gather(jg)

        @pl.when(b + 3 < nbatch)
        def _():
            eload(b + 3, js)

        gather_wait(i)
        scale(i)
        scatter(i)

    for k in range(3):
        eload(k, k)
    for k in range(2):
        eload_wait(k)
        gather(k)

    def quad_body(t, _):
        b0 = 4 * t
        step(b0 + 0, 0)
        step(b0 + 1, 1)
        step(b0 + 2, 2)
        step(b0 + 3, 3)
        return _

    lax.fori_loop(0, nquad, quad_body, None)
    step(nbatch - 1, 0)
    scatter_wait(3)
    scatter_wait(0)

    plsc.subcore_barrier()

    def wb_body(t, _):
        chunk = s + NS * t

        @pl.when(chunk < nchunk_rows)
        def _():
            r0 = chunk * B
            pltpu.sync_copy(acc_sh.at[pl.ds(r0, B)], rows[0])
            pltpu.sync_copy(rows[0], out_hbm.at[c, pl.ds(r0, B)])

        return _

    lax.fori_loop(0, nround, wb_body, None)


@functools.partial(jax.jit, static_argnames=("N", "E", "D"))
def _sc_scatter(x, src, dst, ew, N, E, D):
    B = 80
    NW = NC * NS
    mesh = plsc.VectorSubcoreMesh(
        core_axis_name="c", subcore_axis_name="s",
        num_cores=NC, num_subcores=NS)
    packed = jnp.stack(
        [src.reshape(-1, B), dst.reshape(-1, B),
         lax.bitcast_convert_type(ew, jnp.int32).reshape(-1, B)], axis=1)
    f = pl.kernel(
        functools.partial(_sc_scatter_fn, N, E, D, B),
        out_type=jax.ShapeDtypeStruct((NC, N, D), jnp.float32),
        mesh=mesh,
        scratch_types=[
            [pltpu.VMEM((3, B), jnp.int32) for _ in range(4)],
            [pltpu.VMEM((B,), jnp.int32) for _ in range(4)],
            [pltpu.VMEM((B, D), jnp.float32) for _ in range(4)],
            pltpu.VMEM_SHARED((N, D), jnp.float32),
            [[pltpu.SemaphoreType.DMA for _ in range(4)]
             for _ in range(3)],
        ],
    )
    return f(x, packed)


def _tc_fn(p_ref, w_ref, b_ref, o_ref):
    a = p_ref[0] + p_ref[1]
    o_ref[...] = jnp.dot(a, w_ref[...],
                         preferred_element_type=jnp.float32) + b_ref[...]


@functools.partial(jax.jit, static_argnames=("bn",))
def _tc_finish(partials, weight, bias2d, bn):
    N, D = partials.shape[1], partials.shape[2]
    DO = weight.shape[1]
    grid = (N // bn,)
    return pl.pallas_call(
        _tc_fn,
        grid=grid,
        in_specs=[
            pl.BlockSpec((NC, bn, D), lambda i: (0, i, 0)),
            pl.BlockSpec((D, DO), lambda i: (0, 0)),
            pl.BlockSpec((1, DO), lambda i: (0, 0)),
        ],
        out_specs=pl.BlockSpec((bn, DO), lambda i: (i, 0)),
        out_shape=jax.ShapeDtypeStruct((N, DO), jnp.float32),
    )(partials, weight, bias2d)


def kernel(x, edge_index, edge_weight, weight, bias):
    N, D = x.shape
    E = edge_index.shape[1]
    ew = edge_weight.reshape(-1)
    src = edge_index[0]
    dst = edge_index[1]
    partials = _sc_scatter(x, src, dst, ew, N=N, E=E, D=D)
    return _tc_finish(partials, weight, bias.reshape(1, -1), bn=2000)


# speedup vs baseline: 1.0855x; 1.0855x over previous
import functools
import jax
import jax.numpy as jnp
from jax import lax
from jax.experimental import pallas as pl
from jax.experimental.pallas import tpu as pltpu
from jax.experimental.pallas import tpu_sc as plsc

NC = 2
NS = 16
L = 16


def _sc_scatter_fn(N, E, D, B, x_hbm, src_hbm, dst_hbm, ew_hbm, out_hbm,
                   sv, dv, wv, rows, acc_sh, sems):
    NW = NC * NS
    e_per_tile = E // NW
    nbatch = e_per_tile // B
    nquad = (nbatch - 1) // 4
    nchunk_rows = N // B
    nround = pl.cdiv(nchunk_rows, NS)
    nchunk = D // L

    c = lax.axis_index("c")
    s = lax.axis_index("s")
    wid = s * NC + c

    zeros = jnp.zeros((L,), jnp.float32)

    def zero_body(t, _):
        r = t // nchunk
        j = t % nchunk
        rows[0][r, pl.ds(j * L, L)] = zeros
        return _

    lax.fori_loop(0, B * nchunk, zero_body, None)

    def acczero_body(t, _):
        chunk = s + NS * t

        @pl.when(chunk < nchunk_rows)
        def _():
            pltpu.sync_copy(rows[0], acc_sh.at[pl.ds(chunk * B, B)])

        return _

    lax.fori_loop(0, nround, acczero_body, None)
    plsc.subcore_barrier()

    ebase = wid * e_per_tile
    ssrc, sg, sd, sw, ss = sems

    def eload(hbm, b, v, sem):
        pltpu.async_copy(hbm.at[pl.ds(ebase + b * B, B)], v, sem)

    def eload_wait(hbm, v, sem):
        pltpu.make_async_copy(hbm.at[pl.ds(0, B)], v, sem).wait()

    def gather(k):
        pltpu.async_copy(x_hbm.at[sv[k]], rows[k], sg[k])

    def gather_wait(k):
        pltpu.make_async_copy(x_hbm.at[pl.ds(0, B)], rows[k], sg[k]).wait()

    def scale(k):
        def scale_body(g, _):
            wgrp = wv[k][pl.ds(g * L, L)]
            for i in range(L):
                e = g * L + i
                wvec = lax.gather(
                    wgrp, jnp.full((L, 1), i, jnp.int32),
                    lax.GatherDimensionNumbers(
                        offset_dims=(), collapsed_slice_dims=(0,),
                        start_index_map=(0,)),
                    (1,), mode=lax.GatherScatterMode.PROMISE_IN_BOUNDS)
                for j in range(nchunk):
                    sl = pl.ds(j * L, L)
                    rows[k][e, sl] = rows[k][e, sl] * wvec
            return _

        lax.fori_loop(0, B // L, scale_body, None)

    def scatter(k):
        pltpu.async_copy(rows[k], acc_sh.at[dv[k]], ss[k], add=True)

    def scatter_wait(k):
        pltpu.make_async_copy(rows[k], acc_sh.at[pl.ds(0, B)], ss[k]).wait()

    def step(b, i):
        jg = (i + 2) % 4
        js = (i + 3) % 4

        @pl.when(b >= 2)
        def _():
            scatter_wait(jg)

        @pl.when(b + 2 < nbatch)
        def _():
            eload_wait(src_hbm, sv[jg], ssrc[jg])
            gather(jg)
            eload(dst_hbm, b + 2, dv[jg], sd[jg])
            eload(ew_hbm, b + 2, wv[jg], sw[jg])

        @pl.when(b + 3 < nbatch)
        def _():
            eload(src_hbm, b + 3, sv[js], ssrc[js])

        gather_wait(i)
        eload_wait(ew_hbm, wv[i], sw[i])
        scale(i)
        eload_wait(dst_hbm, dv[i], sd[i])
        scatter(i)

    for k in range(3):
        eload(src_hbm, k, sv[k], ssrc[k])
    for k in range(2):
        eload_wait(src_hbm, sv[k], ssrc[k])
        gather(k)
        eload(dst_hbm, k, dv[k], sd[k])
        eload(ew_hbm, k, wv[k], sw[k])

    def quad_body(t, _):
        b0 = 4 * t
        step(b0 + 0, 0)
        step(b0 + 1, 1)
        step(b0 + 2, 2)
        step(b0 + 3, 3)
        return _

    lax.fori_loop(0, nquad, quad_body, None)
    step(nbatch - 1, 0)
    scatter_wait(3)
    scatter_wait(0)

    plsc.subcore_barrier()

    def wb_body(t, _):
        chunk = s + NS * t

        @pl.when(chunk < nchunk_rows)
        def _():
            r0 = chunk * B
            pltpu.sync_copy(acc_sh.at[pl.ds(r0, B)], rows[0])
            pltpu.sync_copy(rows[0], out_hbm.at[c, pl.ds(r0, B)])

        return _

    lax.fori_loop(0, nround, wb_body, None)


@functools.partial(jax.jit, static_argnames=("N", "E", "D"))
def _sc_scatter(x, src, dst, ew, N, E, D):
    B = 80
    NW = NC * NS
    mesh = plsc.VectorSubcoreMesh(
        core_axis_name="c", subcore_axis_name="s",
        num_cores=NC, num_subcores=NS)
    f = pl.kernel(
        functools.partial(_sc_scatter_fn, N, E, D, B),
        out_type=jax.ShapeDtypeStruct((NC, N, D), jnp.float32),
        mesh=mesh,
        scratch_types=[
            [pltpu.VMEM((B,), jnp.int32) for _ in range(4)],
            [pltpu.VMEM((B,), jnp.int32) for _ in range(4)],
            [pltpu.VMEM((B,), jnp.float32) for _ in range(4)],
            [pltpu.VMEM((B, D), jnp.float32) for _ in range(4)],
            pltpu.VMEM_SHARED((N, D), jnp.float32),
            [[pltpu.SemaphoreType.DMA for _ in range(4)]
             for _ in range(5)],
        ],
    )
    return f(x, src, dst, ew)


def _tc_fn(p_ref, w_ref, b_ref, o_ref):
    a = p_ref[0] + p_ref[1]
    o_ref[...] = jnp.dot(a, w_ref[...],
                         preferred_element_type=jnp.float32) + b_ref[...]


@functools.partial(jax.jit, static_argnames=("bn",))
def _tc_finish(partials, weight, bias2d, bn):
    N, D = partials.shape[1], partials.shape[2]
    DO = weight.shape[1]
    grid = (N // bn,)
    return pl.pallas_call(
        _tc_fn,
        grid=grid,
        in_specs=[
            pl.BlockSpec((NC, bn, D), lambda i: (0, i, 0)),
            pl.BlockSpec((D, DO), lambda i: (0, 0)),
            pl.BlockSpec((1, DO), lambda i: (0, 0)),
        ],
        out_specs=pl.BlockSpec((bn, DO), lambda i: (i, 0)),
        out_shape=jax.ShapeDtypeStruct((N, DO), jnp.float32),
    )(partials, weight, bias2d)


def kernel(x, edge_index, edge_weight, weight, bias):
    N, D = x.shape
    E = edge_index.shape[1]
    ew = edge_weight.reshape(-1)
    src = edge_index[0]
    dst = edge_index[1]
    partials = _sc_scatter(x, src, dst, ew, N=N, E=E, D=D)
    return _tc_finish(partials, weight, bias.reshape(1, -1), bn=2000)


# speedup vs baseline: 1.0910x; 1.0050x over previous
import functools
import jax
import jax.numpy as jnp
from jax import lax
from jax.experimental import pallas as pl
from jax.experimental.pallas import tpu as pltpu
from jax.experimental.pallas import tpu_sc as plsc

NC = 2
NS = 16
L = 16


def _sc_scatter_fn(N, E, D, B, x_hbm, src_hbm, dst_hbm, ew_hbm, out_hbm,
                   sv, dv, wv, rows, acc_sh, sems):
    NW = NC * NS
    e_per_tile = E // NW
    nbatch = e_per_tile // B
    nquad = (nbatch - 1) // 4
    nchunk_rows = N // B
    nround = pl.cdiv(nchunk_rows, NS)
    nchunk = D // L

    c = lax.axis_index("c")
    s = lax.axis_index("s")
    wid = s * NC + c

    zeros = jnp.zeros((L,), jnp.float32)

    def zero_body(t, _):
        r = t // nchunk
        j = t % nchunk
        rows[0][r, pl.ds(j * L, L)] = zeros
        return _

    lax.fori_loop(0, B * nchunk, zero_body, None)

    def acczero_body(t, _):
        chunk = s + NS * t

        @pl.when(chunk < nchunk_rows)
        def _():
            pltpu.sync_copy(rows[0], acc_sh.at[pl.ds(chunk * B, B)])

        return _

    lax.fori_loop(0, nround, acczero_body, None)
    plsc.subcore_barrier()

    ebase = wid * e_per_tile
    ssrc, sg, sd, sw, ss = sems

    def eload(hbm, b, v, sem):
        pltpu.async_copy(hbm.at[pl.ds(ebase + b * B, B)], v, sem)

    def eload_wait(hbm, v, sem):
        pltpu.make_async_copy(hbm.at[pl.ds(0, B)], v, sem).wait()

    def gather(k):
        pltpu.async_copy(x_hbm.at[sv[k]], rows[k], sg[k])

    def gather_wait(k):
        pltpu.make_async_copy(x_hbm.at[pl.ds(0, B)], rows[k], sg[k]).wait()

    def scale(k):
        def scale_body(g, _):
            wgrp = wv[k][pl.ds(g * L, L)]
            for i in range(L):
                e = g * L + i
                wvec = lax.gather(
                    wgrp, jnp.full((L, 1), i, jnp.int32),
                    lax.GatherDimensionNumbers(
                        offset_dims=(), collapsed_slice_dims=(0,),
                        start_index_map=(0,)),
                    (1,), mode=lax.GatherScatterMode.PROMISE_IN_BOUNDS)
                for j in range(nchunk):
                    sl = pl.ds(j * L, L)
                    rows[k][e, sl] = rows[k][e, sl] * wvec
            return _

        lax.fori_loop(0, B // L, scale_body, None)

    def scatter(k):
        pltpu.async_copy(rows[k], acc_sh.at[dv[k]], ss[k], add=True)

    def scatter_wait(k):
        pltpu.make_async_copy(rows[k], acc_sh.at[pl.ds(0, B)], ss[k]).wait()

    def step(b, i):
        jg = (i + 2) % 4
        js = (i + 3) % 4

        @pl.when(b >= 2)
        def _():
            scatter_wait(jg)

        @pl.when(b + 2 < nbatch)
        def _():
            eload_wait(src_hbm, sv[jg], ssrc[jg])
            gather(jg)
            eload(dst_hbm, b + 2, dv[jg], sd[jg])
            eload(ew_hbm, b + 2, wv[jg], sw[jg])

        @pl.when(b + 3 < nbatch)
        def _():
            eload(src_hbm, b + 3, sv[js], ssrc[js])

        gather_wait(i)
        eload_wait(ew_hbm, wv[i], sw[i])
        scale(i)
        eload_wait(dst_hbm, dv[i], sd[i])
        scatter(i)

    for k in range(3):
        eload(src_hbm, k, sv[k], ssrc[k])
    for k in range(2):
        eload_wait(src_hbm, sv[k], ssrc[k])
        gather(k)
        eload(dst_hbm, k, dv[k], sd[k])
        eload(ew_hbm, k, wv[k], sw[k])

    def quad_body(t, _):
        b0 = 4 * t
        step(b0 + 0, 0)
        step(b0 + 1, 1)
        step(b0 + 2, 2)
        step(b0 + 3, 3)
        return _

    lax.fori_loop(0, nquad, quad_body, None)
    step(nbatch - 1, 0)
    scatter_wait(3)
    scatter_wait(0)

    plsc.subcore_barrier()

    def wb_fire(t, _):
        chunk = s + NS * t

        @pl.when(chunk < nchunk_rows)
        def _():
            r0 = chunk * B
            pltpu.async_copy(acc_sh.at[pl.ds(r0, B)],
                             out_hbm.at[c, pl.ds(r0, B)], ss[0])

        return _

    lax.fori_loop(0, nround, wb_fire, None)

    def wb_drain(t, _):
        chunk = s + NS * t

        @pl.when(chunk < nchunk_rows)
        def _():
            pltpu.make_async_copy(acc_sh.at[pl.ds(0, B)],
                                  out_hbm.at[c, pl.ds(0, B)], ss[0]).wait()

        return _

    lax.fori_loop(0, nround, wb_drain, None)


@functools.partial(jax.jit, static_argnames=("N", "E", "D"))
def _sc_scatter(x, src, dst, ew, N, E, D):
    B = 80
    NW = NC * NS
    mesh = plsc.VectorSubcoreMesh(
        core_axis_name="c", subcore_axis_name="s",
        num_cores=NC, num_subcores=NS)
    f = pl.kernel(
        functools.partial(_sc_scatter_fn, N, E, D, B),
        out_type=jax.ShapeDtypeStruct((NC, N, D), jnp.float32),
        mesh=mesh,
        scratch_types=[
            [pltpu.VMEM((B,), jnp.int32) for _ in range(4)],
            [pltpu.VMEM((B,), jnp.int32) for _ in range(4)],
            [pltpu.VMEM((B,), jnp.float32) for _ in range(4)],
            [pltpu.VMEM((B, D), jnp.float32) for _ in range(4)],
            pltpu.VMEM_SHARED((N, D), jnp.float32),
            [[pltpu.SemaphoreType.DMA for _ in range(4)]
             for _ in range(5)],
        ],
    )
    return f(x, src, dst, ew)


def _tc_fn(p_ref, w_ref, b_ref, o_ref):
    a = p_ref[0] + p_ref[1]
    o_ref[...] = jnp.dot(a, w_ref[...],
                         preferred_element_type=jnp.float32) + b_ref[...]


@functools.partial(jax.jit, static_argnames=("bn",))
def _tc_finish(partials, weight, bias2d, bn):
    N, D = partials.shape[1], partials.shape[2]
    DO = weight.shape[1]
    grid = (N // bn,)
    return pl.pallas_call(
        _tc_fn,
        grid=grid,
        in_specs=[
            pl.BlockSpec((NC, bn, D), lambda i: (0, i, 0)),
            pl.BlockSpec((D, DO), lambda i: (0, 0)),
            pl.BlockSpec((1, DO), lambda i: (0, 0)),
        ],
        out_specs=pl.BlockSpec((bn, DO), lambda i: (i, 0)),
        out_shape=jax.ShapeDtypeStruct((N, DO), jnp.float32),
    )(partials, weight, bias2d)


def kernel(x, edge_index, edge_weight, weight, bias):
    N, D = x.shape
    E = edge_index.shape[1]
    ew = edge_weight.reshape(-1)
    src = edge_index[0]
    dst = edge_index[1]
    partials = _sc_scatter(x, src, dst, ew, N=N, E=E, D=D)
    return _tc_finish(partials, weight, bias.reshape(1, -1), bn=2000)


# speedup vs baseline: 1.0992x; 1.0075x over previous
import functools
import jax
import jax.numpy as jnp
from jax import lax
from jax.experimental import pallas as pl
from jax.experimental.pallas import tpu as pltpu
from jax.experimental.pallas import tpu_sc as plsc

NC = 2
NS = 16
L = 16


def _sc_scatter_fn(N, E, D, B, x_hbm, src_hbm, dst_hbm, ew_hbm, out_hbm,
                   sv, dv, wv, rows, acc_sh, sems):
    NW = NC * NS
    e_per_tile = E // NW
    nbatch = e_per_tile // B
    nquad = (nbatch - 1) // 4
    nchunk_rows = N // B
    nround = pl.cdiv(nchunk_rows, NS)
    nchunk = D // L

    c = lax.axis_index("c")
    s = lax.axis_index("s")
    wid = s * NC + c

    zeros = jnp.zeros((L,), jnp.float32)

    def zero_body(t, _):
        r = t // nchunk
        j = t % nchunk
        rows[0][r, pl.ds(j * L, L)] = zeros
        return _

    lax.fori_loop(0, B * nchunk, zero_body, None)

    def acczero_body(t, _):
        chunk = s + NS * t

        @pl.when(chunk < nchunk_rows)
        def _():
            pltpu.sync_copy(rows[0], acc_sh.at[pl.ds(chunk * B, B)])

        return _

    lax.fori_loop(0, nround, acczero_body, None)
    plsc.subcore_barrier()

    ebase = wid * e_per_tile
    ssrc, sg, sd, sw, ss = sems

    def eload(hbm, b, v, sem):
        pltpu.async_copy(hbm.at[pl.ds(ebase + b * B, B)], v, sem)

    def eload_wait(hbm, v, sem):
        pltpu.make_async_copy(hbm.at[pl.ds(0, B)], v, sem).wait()

    def gather(k):
        pltpu.async_copy(x_hbm.at[sv[k]], rows[k], sg[k])

    def gather_wait(k):
        pltpu.make_async_copy(x_hbm.at[pl.ds(0, B)], rows[k], sg[k]).wait()

    def scale(k):
        def scale_body(g, _):
            wgrp = wv[k][pl.ds(g * L, L)]
            for i in range(L):
                e = g * L + i
                wvec = lax.gather(
                    wgrp, jnp.full((L, 1), i, jnp.int32),
                    lax.GatherDimensionNumbers(
                        offset_dims=(), collapsed_slice_dims=(0,),
                        start_index_map=(0,)),
                    (1,), mode=lax.GatherScatterMode.PROMISE_IN_BOUNDS)
                for j in range(nchunk):
                    sl = pl.ds(j * L, L)
                    rows[k][e, sl] = rows[k][e, sl] * wvec
            return _

        lax.fori_loop(0, B // L, scale_body, None)

    def scatter(k):
        pltpu.async_copy(rows[k], acc_sh.at[dv[k]], ss[k], add=True)

    def scatter_wait(k):
        pltpu.make_async_copy(rows[k], acc_sh.at[pl.ds(0, B)], ss[k]).wait()

    def step(b, i):
        jg = (i + 2) % 4
        js = (i + 3) % 4

        @pl.when(b >= 2)
        def _():
            scatter_wait(jg)

        @pl.when(b + 2 < nbatch)
        def _():
            eload_wait(src_hbm, sv[jg], ssrc[jg])
            gather(jg)
            eload(dst_hbm, b + 2, dv[jg], sd[jg])
            eload(ew_hbm, b + 2, wv[jg], sw[jg])

        @pl.when(b + 3 < nbatch)
        def _():
            eload(src_hbm, b + 3, sv[js], ssrc[js])

        gather_wait(i)
        eload_wait(ew_hbm, wv[i], sw[i])
        scale(i)
        eload_wait(dst_hbm, dv[i], sd[i])
        scatter(i)

    for k in range(3):
        eload(src_hbm, k, sv[k], ssrc[k])
    for k in range(2):
        eload_wait(src_hbm, sv[k], ssrc[k])
        gather(k)
        eload(dst_hbm, k, dv[k], sd[k])
        eload(ew_hbm, k, wv[k], sw[k])

    def quad_body(t, _):
        b0 = 4 * t
        step(b0 + 0, 0)
        step(b0 + 1, 1)
        step(b0 + 2, 2)
        step(b0 + 3, 3)
        return _

    lax.fori_loop(0, nquad, quad_body, None)
    step(nbatch - 1, 0)
    scatter_wait(3)
    scatter_wait(0)

    plsc.subcore_barrier()

    def wb_fire(t, _):
        chunk = s + NS * t

        @pl.when(chunk < nchunk_rows)
        def _():
            r0 = chunk * B
            pltpu.async_copy(acc_sh.at[pl.ds(r0, B)],
                             out_hbm.at[c, pl.ds(r0, B)], ss[0])

        return _

    lax.fori_loop(0, nround, wb_fire, None)

    def wb_drain(t, _):
        chunk = s + NS * t

        @pl.when(chunk < nchunk_rows)
        def _():
            pltpu.make_async_copy(acc_sh.at[pl.ds(0, B)],
                                  out_hbm.at[c, pl.ds(0, B)], ss[0]).wait()

        return _

    lax.fori_loop(0, nround, wb_drain, None)


@functools.partial(jax.jit, static_argnames=("N", "E", "D"))
def _sc_scatter(x, src, dst, ew, N, E, D):
    B = 80
    NW = NC * NS
    mesh = plsc.VectorSubcoreMesh(
        core_axis_name="c", subcore_axis_name="s",
        num_cores=NC, num_subcores=NS)
    f = pl.kernel(
        functools.partial(_sc_scatter_fn, N, E, D, B),
        out_type=jax.ShapeDtypeStruct((NC, N, D), jnp.float32),
        mesh=mesh,
        scratch_types=[
            [pltpu.VMEM((B,), jnp.int32) for _ in range(4)],
            [pltpu.VMEM((B,), jnp.int32) for _ in range(4)],
            [pltpu.VMEM((B,), jnp.float32) for _ in range(4)],
            [pltpu.VMEM((B, D), jnp.float32) for _ in range(4)],
            pltpu.VMEM_SHARED((N, D), jnp.float32),
            [[pltpu.SemaphoreType.DMA for _ in range(4)]
             for _ in range(5)],
        ],
    )
    return f(x, src, dst, ew)


def _tc_fn(p_ref, w_ref, b_ref, o_ref):
    a = p_ref[0] + p_ref[1]
    o_ref[...] = jnp.dot(a, w_ref[...],
                         preferred_element_type=jnp.float32) + b_ref[...]


@functools.partial(jax.jit, static_argnames=("bn",))
def _tc_finish(partials, weight, bias2d, bn):
    N, D = partials.shape[1], partials.shape[2]
    DO = weight.shape[1]
    grid = (N // bn,)
    return pl.pallas_call(
        _tc_fn,
        grid=grid,
        in_specs=[
            pl.BlockSpec((NC, bn, D), lambda i: (0, i, 0)),
            pl.BlockSpec((D, DO), lambda i: (0, 0)),
            pl.BlockSpec((1, DO), lambda i: (0, 0)),
        ],
        out_specs=pl.BlockSpec((bn, DO), lambda i: (i, 0)),
        out_shape=jax.ShapeDtypeStruct((N, DO), jnp.float32),
    )(partials, weight, bias2d)


def kernel(x, edge_index, edge_weight, weight, bias):
    N, D = x.shape
    E = edge_index.shape[1]
    ew = edge_weight.reshape(-1)
    src = edge_index[0]
    dst = edge_index[1]
    partials = _sc_scatter(x, src, dst, ew, N=N, E=E, D=D)
    return _tc_finish(partials, weight, bias.reshape(1, -1), bn=10000)
